# baseline (speedup 1.0000x reference)
import jax
import jax.numpy as jnp
from jax import lax
from jax.experimental import pallas as pl
from jax.experimental.pallas import tpu as pltpu
from jax.experimental.pallas import tpu_sc as plsc

N_WORKERS = 32
CHUNK = 80
BE = 2560


def _sc_gather(atom_features, idx):
    R = idx.shape[0]
    D = atom_features.shape[1]
    dt = atom_features.dtype
    per_w = R // N_WORKERS
    n_chunks = per_w // CHUNK
    mesh = plsc.VectorSubcoreMesh(core_axis_name="c", subcore_axis_name="s")

    def body(atom_hbm, idx_hbm, out_hbm, idx_v, rows_v, sem):
        cid = lax.axis_index("c")
        sid = lax.axis_index("s")
        wid = sid * 2 + cid
        base_w = wid * per_w

        def step(j, carry):
            base = base_w + j * CHUNK
            pltpu.sync_copy(idx_hbm.at[pl.ds(base, CHUNK)], idx_v)
            pltpu.async_copy(atom_hbm.at[idx_v], rows_v, sem).wait()
            pltpu.sync_copy(rows_v, out_hbm.at[pl.ds(base, CHUNK)])
            return carry

        lax.fori_loop(0, n_chunks, step, 0)

    k = pl.kernel(
        body,
        out_type=jax.ShapeDtypeStruct((R, D), dt),
        mesh=mesh,
        scratch_types=[
            pltpu.VMEM((CHUNK,), jnp.int32),
            pltpu.VMEM((CHUNK, D), dt),
            pltpu.SemaphoreType.DMA,
        ],
    )
    return k(atom_features, idx)


def _sigmoid(x):
    return 0.5 * jnp.tanh(0.5 * x) + 0.5


def _silu(x):
    return x * _sigmoid(x)


def _tc_mlp_body(x_ref, bd_ref, wab, wc, b1r, w2, b2r, g2w, gb2r, o_ref):
    x = x_ref[...].astype(jnp.bfloat16)
    bd = bd_ref[...].astype(jnp.bfloat16)
    pre = (jnp.dot(x, wab[...], preferred_element_type=jnp.float32)
           + jnp.dot(bd, wc[...], preferred_element_type=jnp.float32)
           + b1r[...])
    dh = pre.shape[1] // 2
    h = _silu(pre[:, :dh]).astype(jnp.bfloat16)
    g = _silu(pre[:, dh:]).astype(jnp.bfloat16)
    h2 = _silu(jnp.dot(h, w2[...], preferred_element_type=jnp.float32) + b2r[...])
    g2 = _sigmoid(jnp.dot(g, g2w[...], preferred_element_type=jnp.float32)
                  + gb2r[...])
    o_ref[...] = h2 * g2


def _tc_mlp(x, bond, Wab, Wc, b1g, W2, b2, G2, gb2):
    E = bond.shape[0]
    DX = x.shape[1]
    DE = bond.shape[1]
    DH2 = Wab.shape[1]
    DH = W2.shape[0]
    DO = W2.shape[1]
    grid = (E // BE,)

    def blk(shape):
        return pl.BlockSpec(shape, lambda i: (i, 0))

    def full(shape):
        return pl.BlockSpec(shape, lambda i: (0, 0))

    return pl.pallas_call(
        _tc_mlp_body,
        grid=grid,
        in_specs=[
            blk((BE, DX)), blk((BE, DE)),
            full((DX, DH2)), full((DE, DH2)), full((1, DH2)),
            full((DH, DO)), full((1, DO)),
            full((DH, DO)), full((1, DO)),
        ],
        out_specs=blk((BE, DO)),
        out_shape=jax.ShapeDtypeStruct((E, DO), jnp.float32),
    )(x, bond, Wab, Wc, b1g, W2, b2, G2, gb2)


def kernel(atom_features, bond_features, bond_atom_indices,
           W1, b1, W2, b2, G1, gb1, G2, gb2):
    D = atom_features.shape[1]
    E = bond_features.shape[0]
    idx_flat = bond_atom_indices.reshape(-1)
    rows = _sc_gather(atom_features, idx_flat)
    x = rows.reshape(E, 2 * D)
    bf = jnp.bfloat16
    Wab = jnp.concatenate([W1[:2 * D], G1[:2 * D]], axis=1).astype(bf)
    Wc = jnp.concatenate([W1[2 * D:], G1[2 * D:]], axis=1).astype(bf)
    b1g = jnp.concatenate([b1, gb1])[None, :]
    return _tc_mlp(x, bond_features, Wab, Wc, b1g,
                   W2.astype(bf), b2[None, :], G2.astype(bf), gb2[None, :])


# speedup vs baseline: 1.2104x; 1.2104x over previous
import jax
import jax.numpy as jnp
from jax import lax
from jax.experimental import pallas as pl
from jax.experimental.pallas import tpu as pltpu
from jax.experimental.pallas import tpu_sc as plsc

N_WORKERS = 32
CHUNK = 80
NBUF = 5
BE = 2560


def _sc_gather(atom_features, idx):
    R = idx.shape[0]
    D = atom_features.shape[1]
    dt = atom_features.dtype
    per_w = R // N_WORKERS
    n_chunks = per_w // CHUNK
    mesh = plsc.VectorSubcoreMesh(core_axis_name="c", subcore_axis_name="s")

    assert n_chunks % NBUF == 0
    n_outer = n_chunks // NBUF

    def body(atom_hbm, idx_hbm, out_hbm, idx_v, rows_v, *sems):
        sem_i = sems[:NBUF]
        sem_g = sems[NBUF:2 * NBUF]
        sem_s = sems[2 * NBUF:]
        cid = lax.axis_index("c")
        sid = lax.axis_index("s")
        wid = sid * 2 + cid
        base_w = wid * per_w

        def step(k, carry):
            base0 = base_w + k * NBUF * CHUNK
            for b in range(NBUF):
                pltpu.async_copy(idx_hbm.at[pl.ds(base0 + b * CHUNK, CHUNK)],
                                 idx_v.at[b], sem_i[b])
            gathers = []
            for b in range(NBUF):
                pltpu.make_async_copy(idx_hbm.at[pl.ds(base0 + b * CHUNK, CHUNK)],
                                      idx_v.at[b], sem_i[b]).wait()
                gathers.append(
                    pltpu.async_copy(atom_hbm.at[idx_v.at[b]], rows_v.at[b],
                                     sem_g[b]))
            stores = []
            for b in range(NBUF):
                gathers[b].wait()
                stores.append(
                    pltpu.async_copy(rows_v.at[b],
                                     out_hbm.at[pl.ds(base0 + b * CHUNK, CHUNK)],
                                     sem_s[b]))
            for b in range(NBUF):
                stores[b].wait()
            return carry

        lax.fori_loop(0, n_outer, step, 0)

    k = pl.kernel(
        body,
        out_type=jax.ShapeDtypeStruct((R, D), dt),
        mesh=mesh,
        scratch_types=(
            [pltpu.VMEM((NBUF, CHUNK), jnp.int32),
             pltpu.VMEM((NBUF, CHUNK, D), dt)]
            + [pltpu.SemaphoreType.DMA] * (3 * NBUF)
        ),
    )
    return k(atom_features, idx)


def _sigmoid(x):
    return 0.5 * jnp.tanh(0.5 * x) + 0.5


def _silu(x):
    return x * _sigmoid(x)


def _tc_mlp_body(x_ref, bd_ref, wab, wc, b1r, w2, b2r, g2w, gb2r, o_ref):
    x = x_ref[...].astype(jnp.bfloat16)
    bd = bd_ref[...].astype(jnp.bfloat16)
    pre = (jnp.dot(x, wab[...], preferred_element_type=jnp.float32)
           + jnp.dot(bd, wc[...], preferred_element_type=jnp.float32)
           + b1r[...])
    dh = pre.shape[1] // 2
    h = _silu(pre[:, :dh]).astype(jnp.bfloat16)
    g = _silu(pre[:, dh:]).astype(jnp.bfloat16)
    h2 = _silu(jnp.dot(h, w2[...], preferred_element_type=jnp.float32) + b2r[...])
    g2 = _sigmoid(jnp.dot(g, g2w[...], preferred_element_type=jnp.float32)
                  + gb2r[...])
    o_ref[...] = h2 * g2


def _tc_mlp(x, bond, Wab, Wc, b1g, W2, b2, G2, gb2):
    E = bond.shape[0]
    DX = x.shape[1]
    DE = bond.shape[1]
    DH2 = Wab.shape[1]
    DH = W2.shape[0]
    DO = W2.shape[1]
    grid = (E // BE,)

    def blk(shape):
        return pl.BlockSpec(shape, lambda i: (i, 0))

    def full(shape):
        return pl.BlockSpec(shape, lambda i: (0, 0))

    return pl.pallas_call(
        _tc_mlp_body,
        grid=grid,
        in_specs=[
            blk((BE, DX)), blk((BE, DE)),
            full((DX, DH2)), full((DE, DH2)), full((1, DH2)),
            full((DH, DO)), full((1, DO)),
            full((DH, DO)), full((1, DO)),
        ],
        out_specs=blk((BE, DO)),
        out_shape=jax.ShapeDtypeStruct((E, DO), jnp.float32),
    )(x, bond, Wab, Wc, b1g, W2, b2, G2, gb2)


def kernel(atom_features, bond_features, bond_atom_indices,
           W1, b1, W2, b2, G1, gb1, G2, gb2):
    D = atom_features.shape[1]
    E = bond_features.shape[0]
    idx_flat = bond_atom_indices.reshape(-1)
    rows = _sc_gather(atom_features, idx_flat)
    x = rows.reshape(E, 2 * D)
    bf = jnp.bfloat16
    Wab = jnp.concatenate([W1[:2 * D], G1[:2 * D]], axis=1).astype(bf)
    Wc = jnp.concatenate([W1[2 * D:], G1[2 * D:]], axis=1).astype(bf)
    b1g = jnp.concatenate([b1, gb1])[None, :]
    return _tc_mlp(x, bond_features, Wab, Wc, b1g,
                   W2.astype(bf), b2[None, :], G2.astype(bf), gb2[None, :])


# speedup vs baseline: 2.2225x; 1.8361x over previous
import jax
import jax.numpy as jnp
from jax import lax
from jax.experimental import pallas as pl
from jax.experimental.pallas import tpu as pltpu
from jax.experimental.pallas import tpu_sc as plsc

N_WORKERS = 32
CHUNK = 80
NBUF = 5
BE = 2560


def _sc_gather_concat(atom_features, idx0, idx1):
    E = idx0.shape[0]
    D = atom_features.shape[1]
    dt = atom_features.dtype
    per_w = E // N_WORKERS
    n_chunks = per_w // CHUNK
    assert n_chunks % NBUF == 0
    n_outer = n_chunks // NBUF
    mesh = plsc.VectorSubcoreMesh(core_axis_name="c", subcore_axis_name="s")

    def body(atom_hbm, idx0_hbm, idx1_hbm, out_hbm, idx_v, rows_v, *sems):
        sem_i = sems[:2 * NBUF]
        sem_g = sems[2 * NBUF:4 * NBUF]
        sem_s = sems[4 * NBUF:]
        cid = lax.axis_index("c")
        sid = lax.axis_index("s")
        wid = sid * 2 + cid
        base_w = wid * per_w

        def step(k, carry):
            base0 = base_w + k * NBUF * CHUNK
            idx_srcs = []
            for b in range(NBUF):
                for h, idx_hbm in enumerate((idx0_hbm, idx1_hbm)):
                    src = idx_hbm.at[pl.ds(base0 + b * CHUNK, CHUNK)]
                    idx_srcs.append(src)
                    pltpu.async_copy(src, idx_v.at[2 * b + h], sem_i[2 * b + h])
            gathers = []
            for u in range(2 * NBUF):
                pltpu.make_async_copy(idx_srcs[u], idx_v.at[u], sem_i[u]).wait()
                gathers.append(
                    pltpu.async_copy(atom_hbm.at[idx_v.at[u]], rows_v.at[u],
                                     sem_g[u]))
            stores = []
            for b in range(NBUF):
                for h in range(2):
                    u = 2 * b + h
                    gathers[u].wait()
                    stores.append(
                        pltpu.async_copy(
                            rows_v.at[u],
                            out_hbm.at[pl.ds(base0 + b * CHUNK, CHUNK),
                                       pl.ds(h * D, D)],
                            sem_s[u]))
            for st in stores:
                st.wait()
            return carry

        lax.fori_loop(0, n_outer, step, 0)

    k = pl.kernel(
        body,
        out_type=jax.ShapeDtypeStruct((E, 2 * D), dt),
        mesh=mesh,
        scratch_types=(
            [pltpu.VMEM((2 * NBUF, CHUNK), jnp.int32),
             pltpu.VMEM((2 * NBUF, CHUNK, D), dt)]
            + [pltpu.SemaphoreType.DMA] * (6 * NBUF)
        ),
    )
    return k(atom_features, idx0, idx1)


def _tc_mlp_body(x_ref, bd_ref, wab, wc, b1r, w2, b2r, g2w, gb2r, o_ref):
    x = x_ref[...].astype(jnp.bfloat16)
    bd = bd_ref[...].astype(jnp.bfloat16)
    t = (jnp.dot(x, wab[...], preferred_element_type=jnp.float32)
         + jnp.dot(bd, wc[...], preferred_element_type=jnp.float32)
         + b1r[...])
    act = t + t * jnp.tanh(t)
    dh = act.shape[1] // 2
    h = act[:, :dh].astype(jnp.bfloat16)
    g = act[:, dh:].astype(jnp.bfloat16)
    t2 = jnp.dot(h, w2[...], preferred_element_type=jnp.float32) + b2r[...]
    h2 = t2 + t2 * jnp.tanh(t2)
    tg = jnp.dot(g, g2w[...], preferred_element_type=jnp.float32) + gb2r[...]
    o_ref[...] = 0.5 * (h2 + h2 * jnp.tanh(tg))


def _tc_mlp(x, bond, Wab, Wc, b1g, W2, b2, G2, gb2):
    E = bond.shape[0]
    DX = x.shape[1]
    DE = bond.shape[1]
    DH2 = Wab.shape[1]
    DH = W2.shape[0]
    DO = W2.shape[1]
    grid = (E // BE,)

    def blk(shape):
        return pl.BlockSpec(shape, lambda i: (i, 0))

    def full(shape):
        return pl.BlockSpec(shape, lambda i: (0, 0))

    return pl.pallas_call(
        _tc_mlp_body,
        grid=grid,
        in_specs=[
            blk((BE, DX)), blk((BE, DE)),
            full((DX, DH2)), full((DE, DH2)), full((1, DH2)),
            full((DH, DO)), full((1, DO)),
            full((DH, DO)), full((1, DO)),
        ],
        out_specs=blk((BE, DO)),
        out_shape=jax.ShapeDtypeStruct((E, DO), jnp.float32),
    )(x, bond, Wab, Wc, b1g, W2, b2, G2, gb2)


def kernel(atom_features, bond_features, bond_atom_indices,
           W1, b1, W2, b2, G1, gb1, G2, gb2):
    D = atom_features.shape[1]
    idx0 = bond_atom_indices[:, 0]
    idx1 = bond_atom_indices[:, 1]
    x = _sc_gather_concat(atom_features, idx0, idx1)
    bf = jnp.bfloat16
    Wab = (0.5 * jnp.concatenate([W1[:2 * D], G1[:2 * D]], axis=1)).astype(bf)
    Wc = (0.5 * jnp.concatenate([W1[2 * D:], G1[2 * D:]], axis=1)).astype(bf)
    b1g = 0.5 * jnp.concatenate([b1, gb1])[None, :]
    return _tc_mlp(x, bond_features, Wab, Wc, b1g,
                   (0.5 * W2).astype(bf), 0.5 * b2[None, :],
                   (0.5 * G2).astype(bf), 0.5 * gb2[None, :])
